# Initial kernel scaffold; baseline (speedup 1.0000x reference)
import functools

import jax
import jax.numpy as jnp
from jax import lax
from jax.experimental import pallas as pl
from jax.experimental.pallas import tpu as pltpu
from jax.experimental.pallas import tpu_sc as plsc

NUM_INST = 1000000
OUT_SIZE = 32
B, L = 16384, 50

NC, NS, LANES = 2, 16, 16
NW = NC * NS
N = B * L
PER_W = N // NW
CHUNK = 1280
NCHUNK = PER_W // CHUNK
SUB = 128
NSUB = CHUNK // SUB

_mesh = plsc.VectorSubcoreMesh(core_axis_name="c", subcore_axis_name="s")


@functools.partial(
    pl.kernel,
    out_type=jax.ShapeDtypeStruct((N, OUT_SIZE), jnp.float32),
    mesh=_mesh,
    scratch_types=[
        pltpu.VMEM((CHUNK,), jnp.int32),
        pltpu.VMEM((CHUNK, OUT_SIZE), jnp.float32),
        pltpu.SemaphoreType.DMA,
    ],
)
def _sc_gather(x_hbm, table_hbm, out_hbm, idx_v, rows_v, sem):
    wid = lax.axis_index("s") * NC + lax.axis_index("c")
    base = wid * PER_W

    @pl.loop(0, NCHUNK)
    def _chunk(ci):
        off = base + ci * CHUNK
        pltpu.sync_copy(x_hbm.at[pl.ds(off, CHUNK)], idx_v)

        descs = [
            pltpu.async_copy(
                table_hbm.at[idx_v.at[pl.ds(j * SUB, SUB)]],
                rows_v.at[pl.ds(j * SUB, SUB), :],
                sem,
            )
            for j in range(NSUB)
        ]

        def _mx_body(g, acc):
            return jnp.maximum(acc, idx_v[pl.ds(g * LANES, LANES)])

        mx = lax.fori_loop(
            0, CHUNK // LANES, _mx_body, jnp.zeros((LANES,), jnp.int32)
        )
        has_pad = jnp.max(mx) >= NUM_INST

        for d in descs:
            d.wait()

        @pl.when(has_pad)
        def _fixup():
            @pl.loop(0, CHUNK // LANES)
            def _grp(g):
                v = idx_v[pl.ds(g * LANES, LANES)]
                m = v >= NUM_INST

                @pl.when(jnp.max(v) >= NUM_INST)
                def _zero_rows():
                    rows = (
                        lax.broadcasted_iota(jnp.int32, (LANES,), 0)
                        + g * LANES
                    )
                    z = jnp.zeros((LANES,), jnp.float32)
                    for c in range(OUT_SIZE):
                        plsc.store_scatter(
                            rows_v,
                            [rows, jnp.full((LANES,), c, jnp.int32)],
                            z,
                            mask=m,
                        )

        pltpu.sync_copy(rows_v, out_hbm.at[pl.ds(off, CHUNK), :])


@jax.jit
def kernel(x, _, table):
    xf = x.reshape(-1).astype(jnp.int32)
    out = _sc_gather(xf, table)
    return out.reshape(B, L, OUT_SIZE)


# speedup vs baseline: 1.1492x; 1.1492x over previous
import functools

import jax
import jax.numpy as jnp
from jax import lax
from jax.experimental import pallas as pl
from jax.experimental.pallas import tpu as pltpu
from jax.experimental.pallas import tpu_sc as plsc

NUM_INST = 1000000
OUT_SIZE = 32
B, L = 16384, 50

NC, NS, LANES = 2, 16, 16
NW = NC * NS
N = B * L
PER_W = N // NW
CHUNK = 1280
NCHUNK = PER_W // CHUNK
SUB = 128
NSUB = CHUNK // SUB

_mesh = plsc.VectorSubcoreMesh(core_axis_name="c", subcore_axis_name="s")


@functools.partial(
    pl.kernel,
    out_type=jax.ShapeDtypeStruct((N, OUT_SIZE), jnp.float32),
    mesh=_mesh,
    scratch_types=[
        pltpu.VMEM((CHUNK,), jnp.int32),
        pltpu.VMEM((CHUNK, OUT_SIZE), jnp.float32),
        pltpu.SemaphoreType.DMA,
    ],
    compiler_params=pltpu.CompilerParams(use_tc_tiling_on_sc=False),
)
def _sc_gather(x_hbm, table_hbm, out_hbm, idx_v, rows_v, sem):
    wid = lax.axis_index("s") * NC + lax.axis_index("c")
    base = wid * PER_W

    @pl.loop(0, NCHUNK)
    def _chunk(ci):
        off = base + ci * CHUNK
        pltpu.sync_copy(x_hbm.at[pl.ds(off, CHUNK)], idx_v)

        descs = [
            pltpu.async_copy(
                table_hbm.at[idx_v.at[pl.ds(j * SUB, SUB)]],
                rows_v.at[pl.ds(j * SUB, SUB), :],
                sem,
            )
            for j in range(NSUB)
        ]

        def _mx_body(g, acc):
            return jnp.maximum(acc, idx_v[pl.ds(g * LANES, LANES)])

        mx = lax.fori_loop(
            0, CHUNK // LANES, _mx_body, jnp.zeros((LANES,), jnp.int32)
        )
        smx = mx[0]
        for i in range(1, LANES):
            smx = jnp.maximum(smx, mx[i])

        for d in descs:
            d.wait()

        @pl.when(smx >= NUM_INST)
        def _fixup():
            @pl.loop(0, CHUNK // LANES)
            def _grp(g):
                vg = idx_v[pl.ds(g * LANES, LANES)]
                for j in range(LANES):
                    r = g * LANES + j
                    scale = jnp.where(
                        vg[j] >= NUM_INST, jnp.float32(0.0), jnp.float32(1.0)
                    )
                    rows_v[r, pl.ds(0, LANES)] = (
                        rows_v[r, pl.ds(0, LANES)] * scale
                    )
                    rows_v[r, pl.ds(LANES, LANES)] = (
                        rows_v[r, pl.ds(LANES, LANES)] * scale
                    )

        pltpu.sync_copy(rows_v, out_hbm.at[pl.ds(off, CHUNK), :])


@jax.jit
def kernel(x, _, table):
    xf = x.reshape(-1).astype(jnp.int32)
    out = _sc_gather(xf, table)
    return out.reshape(B, L, OUT_SIZE)
